# baseline (speedup 1.0000x reference)
import functools

import jax
import jax.numpy as jnp
from jax.experimental import pallas as pl
from jax.experimental.pallas import tpu as pltpu

_NUM_CLASSES = 100000
_FEAT_LEN = 64
_BATCH = 1024
_BN = 4096


def _matmul_kernel(feats_ref, pid_ref, lut_ref, score_ref, labels_ref):
    f = feats_ref[...].astype(jnp.bfloat16)
    w = lut_ref[...].astype(jnp.bfloat16)
    score_ref[...] = jax.lax.dot_general(
        f, w, (((1,), (1,)), ((), ())), preferred_element_type=jnp.float32
    )
    p = pid_ref[...]
    labels_ref[...] = jnp.where((p < 0) | (p >= _NUM_CLASSES), -1, p)


@functools.partial(jax.jit, static_argnames=())
def kernel(feats, pid_labels, lookup_table):
    pid2d = pid_labels.reshape(8, 128)
    grid = (pl.cdiv(_NUM_CLASSES, _BN),)
    score, labels2d = pl.pallas_call(
        _matmul_kernel,
        grid=grid,
        in_specs=[
            pl.BlockSpec((_BATCH, _FEAT_LEN), lambda i: (0, 0)),
            pl.BlockSpec((8, 128), lambda i: (0, 0)),
            pl.BlockSpec((_BN, _FEAT_LEN), lambda i: (i, 0)),
        ],
        out_specs=[
            pl.BlockSpec((_BATCH, _BN), lambda i: (0, i)),
            pl.BlockSpec((8, 128), lambda i: (0, 0)),
        ],
        out_shape=[
            jax.ShapeDtypeStruct((_BATCH, _NUM_CLASSES), jnp.float32),
            jax.ShapeDtypeStruct((8, 128), jnp.int32),
        ],
        compiler_params=pltpu.CompilerParams(
            dimension_semantics=("parallel",),
        ),
    )(feats, pid2d, lookup_table)
    return (score, labels2d.reshape(-1))


# speedup vs baseline: 1.0004x; 1.0004x over previous
import jax
import jax.numpy as jnp
from jax.experimental import pallas as pl
from jax.experimental.pallas import tpu as pltpu

_NUM_CLASSES = 100000
_FEAT_LEN = 64
_BATCH = 1024
_BN = 4096
_NFULL = _NUM_CLASSES // _BN
_TAIL = _NUM_CLASSES - _NFULL * _BN
_NSTEPS = _NFULL + 1
_NSPLIT = 8
_RB = _BATCH // _NSPLIT
_TAIL_A = (_TAIL // 128) * 128
_TAIL_B = _TAIL - _TAIL_A


def _copies(scratch, hbm_out, sems, slot, col, width, nrows=_RB):
    return [
        pltpu.make_async_copy(
            scratch.at[slot, pl.ds(r * nrows, nrows), pl.ds(0, width)],
            hbm_out.at[pl.ds(r * nrows, nrows), pl.ds(col, width)],
            sems.at[slot, r],
        )
        for r in range(_NSPLIT)
    ]


def _tail_copies(scratch, tail32, hbm_out, sems, tail_sem, slot):
    col = _NFULL * _BN
    cps = _copies(scratch, hbm_out, sems, slot, col, _TAIL_A)
    cps.append(
        pltpu.make_async_copy(
            tail32,
            hbm_out.at[:, pl.ds(col + _TAIL_A, _TAIL_B)],
            tail_sem,
        )
    )
    return cps


def _mm_kernel(feats_ref, pid_ref, lut_ref, hbm_out, labels_ref, scratch,
               tail32, sems, tail_sem):
    i = pl.program_id(0)
    slot = jax.lax.rem(i, 2)
    f = feats_ref[...].astype(jnp.bfloat16)
    w = lut_ref[...].astype(jnp.bfloat16)
    scratch[slot] = jax.lax.dot_general(
        f, w, (((1,), (1,)), ((), ())), preferred_element_type=jnp.float32
    )

    @pl.when(i < _NFULL)
    def _start_full():
        for c in _copies(scratch, hbm_out, sems, slot, i * _BN, _BN):
            c.start()

    @pl.when(i == _NFULL)
    def _start_tail():
        w_tail = w[_TAIL_A:_TAIL_A + _TAIL_B, :]
        tail32[...] = jax.lax.dot_general(
            f, w_tail, (((1,), (1,)), ((), ())),
            preferred_element_type=jnp.float32,
        )
        for c in _tail_copies(scratch, tail32, hbm_out, sems, tail_sem, slot):
            c.start()

    @pl.when(i > 0)
    def _wait_other():
        for c in _copies(scratch, hbm_out, sems, 1 - slot, (i - 1) * _BN, _BN):
            c.wait()

    @pl.when(i == _NFULL)
    def _wait_tail():
        for c in _tail_copies(scratch, tail32, hbm_out, sems, tail_sem, slot):
            c.wait()

    p = pid_ref[...]
    labels_ref[...] = jnp.where((p < 0) | (p >= _NUM_CLASSES), -1, p)


def kernel(feats, pid_labels, lookup_table):
    pid2d = pid_labels.reshape(8, 128)
    score, labels2d = pl.pallas_call(
        _mm_kernel,
        grid=(_NSTEPS,),
        in_specs=[
            pl.BlockSpec((_BATCH, _FEAT_LEN), lambda i: (0, 0)),
            pl.BlockSpec((8, 128), lambda i: (0, 0)),
            pl.BlockSpec((_BN, _FEAT_LEN), lambda i: (i, 0)),
        ],
        out_specs=[
            pl.BlockSpec(memory_space=pltpu.MemorySpace.HBM),
            pl.BlockSpec((8, 128), lambda i: (0, 0)),
        ],
        out_shape=[
            jax.ShapeDtypeStruct((_BATCH, _NUM_CLASSES), jnp.float32),
            jax.ShapeDtypeStruct((8, 128), jnp.int32),
        ],
        scratch_shapes=[
            pltpu.VMEM((2, _BATCH, _BN), jnp.float32),
            pltpu.VMEM((_BATCH, _TAIL_B), jnp.float32),
            pltpu.SemaphoreType.DMA((2, _NSPLIT)),
            pltpu.SemaphoreType.DMA(()),
        ],
        compiler_params=pltpu.CompilerParams(
            dimension_semantics=("arbitrary",),
        ),
    )(feats, pid2d, lookup_table)
    return (score, labels2d.reshape(-1))
